# baseline (speedup 1.0000x reference)
import functools

import jax
import jax.numpy as jnp
from jax import lax
from jax.experimental import pallas as pl
from jax.experimental.pallas import tpu as pltpu
from jax.experimental.pallas import tpu_sc as plsc

B, L, T = 8, 80, 256
C, NQ, NS, D = 365, 100, 4, 256
NQSEL = 5
RD = NS * D
NROWS = B * L * NQSEL
NW = 32
BPW_HI = 104
BPW_LO = 96
SPLIT = 16 * BPW_HI
REP = NQSEL * NS


def _sc_gather_rows(bank3, idx1d):
    mesh = plsc.VectorSubcoreMesh(core_axis_name="c", subcore_axis_name="s")

    @functools.partial(
        pl.kernel,
        mesh=mesh,
        out_type=jax.ShapeDtypeStruct((NROWS * NS, D), jnp.float32),
        scratch_types=[
            pltpu.VMEM((BPW_HI,), jnp.int32),
            pltpu.VMEM((BPW_HI, NS, D), jnp.float32),
            pltpu.SemaphoreType.DMA,
        ],
    )
    def k(table_hbm, idx_hbm, out_hbm, idx_v, rows_v, sem):
        wid = lax.axis_index("s") * 2 + lax.axis_index("c")
        pltpu.sync_copy(idx_hbm.at[pl.ds(pl.multiple_of(wid * BPW_HI, 8),
                                         BPW_HI)], idx_v)
        pltpu.async_copy(table_hbm.at[idx_v], rows_v, sem).wait()
        flat = rows_v.reshape(BPW_HI * NS, D)

        @pl.when(wid < 16)
        def _hi():
            base = pl.multiple_of(wid * BPW_HI * NS, 8)
            pltpu.sync_copy(flat, out_hbm.at[pl.ds(base, BPW_HI * NS)])

        @pl.when(wid >= 16)
        def _lo():
            base = pl.multiple_of((SPLIT + (wid - 16) * BPW_LO) * NS, 8)
            pltpu.sync_copy(flat.at[pl.ds(0, BPW_LO * NS)],
                            out_hbm.at[pl.ds(base, BPW_LO * NS)])

    return k(bank3, idx1d)


MGRP = 8


def _tc_mask_body(loc_ref, out_ref):
    y = (loc_ref[...] != 0.0).astype(jnp.float32)
    ridx = jax.lax.broadcasted_iota(jnp.int32, (MGRP * REP, T), 0) // REP
    out_ref[...] = jnp.take_along_axis(y, ridx, axis=0)


def _tc_mask(loc2):
    return pl.pallas_call(
        _tc_mask_body,
        grid=(B * L // MGRP,),
        in_specs=[pl.BlockSpec((MGRP, T), lambda i: (i, 0))],
        out_specs=pl.BlockSpec((MGRP * REP, T), lambda i: (i, 0)),
        out_shape=jax.ShapeDtypeStruct((B * L * REP, T), jnp.float32),
    )(loc2)


def kernel(bank, loc_map, labels, sel_idx):
    bank3 = bank.reshape(C * NQ, NS, D)
    flat_idx = (labels.astype(jnp.int32) * NQ)[:, :, None] + sel_idx.astype(jnp.int32)
    flat_idx = flat_idx.reshape(NROWS)
    hi = flat_idx[:SPLIT].reshape(16, BPW_HI)
    lo = jnp.pad(flat_idx[SPLIT:].reshape(16, BPW_LO),
                 ((0, 0), (0, BPW_HI - BPW_LO)))
    idx1d = jnp.concatenate([hi, lo]).reshape(NW * BPW_HI)

    rows = _sc_gather_rows(bank3, idx1d)
    batched_queries = rows.reshape(B, L * NQSEL * NS, D)

    batched_mask = _tc_mask(loc_map.reshape(B * L, T)).reshape(B, L * REP, T)

    batched_has_vision_query = jnp.ones((B, L), dtype=jnp.int32)
    return batched_queries, batched_mask, batched_has_vision_query


# speedup vs baseline: 1.4491x; 1.4491x over previous
import functools

import jax
import jax.numpy as jnp
from jax import lax
from jax.experimental import pallas as pl
from jax.experimental.pallas import tpu as pltpu
from jax.experimental.pallas import tpu_sc as plsc

B, L, T = 8, 80, 256
C, NQ, NS, D = 365, 100, 4, 256
NQSEL = 5
RD = NS * D
NROWS = B * L * NQSEL
NW = 32
BPW_HI = 104
BPW_LO = 96
SPLIT = 16 * BPW_HI
REP = NQSEL * NS


def _sc_gather_rows(bank3, idx1d):
    mesh = plsc.VectorSubcoreMesh(core_axis_name="c", subcore_axis_name="s")

    @functools.partial(
        pl.kernel,
        mesh=mesh,
        out_type=jax.ShapeDtypeStruct((NROWS * NS, D), jnp.float32),
        scratch_types=[
            pltpu.VMEM((BPW_HI,), jnp.int32),
            pltpu.VMEM((BPW_HI, NS, D), jnp.float32),
            pltpu.SemaphoreType.DMA,
        ],
    )
    def k(table_hbm, idx_hbm, out_hbm, idx_v, rows_v, sem):
        wid = lax.axis_index("s") * 2 + lax.axis_index("c")
        pltpu.sync_copy(idx_hbm.at[pl.ds(pl.multiple_of(wid * BPW_HI, 8),
                                         BPW_HI)], idx_v)
        pltpu.async_copy(table_hbm.at[idx_v], rows_v, sem).wait()
        flat = rows_v.reshape(BPW_HI * NS, D)

        @pl.when(wid < 16)
        def _hi():
            base = pl.multiple_of(wid * BPW_HI * NS, 8)
            pltpu.sync_copy(flat, out_hbm.at[pl.ds(base, BPW_HI * NS)])

        @pl.when(wid >= 16)
        def _lo():
            base = pl.multiple_of((SPLIT + (wid - 16) * BPW_LO) * NS, 8)
            pltpu.sync_copy(flat.at[pl.ds(0, BPW_LO * NS)],
                            out_hbm.at[pl.ds(base, BPW_LO * NS)])

    return k(bank3, idx1d)


MGRP = 80


def _tc_mask_body(loc_ref, out_ref):
    y = (loc_ref[...] != 0.0).astype(jnp.float32)
    rows = jax.lax.broadcasted_iota(jnp.int32, (MGRP * REP, MGRP), 0)
    cols = jax.lax.broadcasted_iota(jnp.int32, (MGRP * REP, MGRP), 1)
    rep = (rows // REP == cols).astype(jnp.float32)
    out_ref[...] = jnp.dot(rep, y, preferred_element_type=jnp.float32)


def _tc_mask(loc2):
    return pl.pallas_call(
        _tc_mask_body,
        grid=(B * L // MGRP,),
        in_specs=[pl.BlockSpec((MGRP, T), lambda i: (i, 0))],
        out_specs=pl.BlockSpec((MGRP * REP, T), lambda i: (i, 0)),
        out_shape=jax.ShapeDtypeStruct((B * L * REP, T), jnp.float32),
    )(loc2)


def kernel(bank, loc_map, labels, sel_idx):
    bank3 = bank.reshape(C * NQ, NS, D)
    flat_idx = (labels.astype(jnp.int32) * NQ)[:, :, None] + sel_idx.astype(jnp.int32)
    flat_idx = flat_idx.reshape(NROWS)
    hi = flat_idx[:SPLIT].reshape(16, BPW_HI)
    lo = jnp.pad(flat_idx[SPLIT:].reshape(16, BPW_LO),
                 ((0, 0), (0, BPW_HI - BPW_LO)))
    idx1d = jnp.concatenate([hi, lo]).reshape(NW * BPW_HI)

    rows = _sc_gather_rows(bank3, idx1d)
    batched_queries = rows.reshape(B, L * NQSEL * NS, D)

    batched_mask = _tc_mask(loc_map.reshape(B * L, T)).reshape(B, L * REP, T)

    batched_has_vision_query = jnp.ones((B, L), dtype=jnp.int32)
    return batched_queries, batched_mask, batched_has_vision_query


# speedup vs baseline: 1.9348x; 1.3351x over previous
import functools

import jax
import jax.numpy as jnp
from jax import lax
from jax.experimental import pallas as pl
from jax.experimental.pallas import tpu as pltpu
from jax.experimental.pallas import tpu_sc as plsc

B, L, T = 8, 80, 256
C, NQ, NS, D = 365, 100, 4, 256
NQSEL = 5
RD = NS * D
NROWS = B * L * NQSEL
NW = 32
BPW = NROWS // NW
REP = NQSEL * NS
CHUNKS = (26, 26, 26, 22)
CHMAX = max(CHUNKS)


def _sc_gather_rows(bank3, idx3):
    mesh = plsc.VectorSubcoreMesh(core_axis_name="c", subcore_axis_name="s")

    @functools.partial(
        pl.kernel,
        mesh=mesh,
        out_type=jax.ShapeDtypeStruct((NROWS * NS, D), jnp.float32),
        scratch_types=[
            pltpu.VMEM((1, BPW), jnp.int32),
            pltpu.VMEM((CHMAX, NS, D), jnp.float32),
            pltpu.VMEM((CHMAX, NS, D), jnp.float32),
            pltpu.SemaphoreType.DMA,
            pltpu.SemaphoreType.DMA,
        ],
    )
    def k(table_hbm, idx_hbm, out_hbm, idx_v, buf0, buf1, gsem, wsem):
        wid = lax.axis_index("s") * 2 + lax.axis_index("c")
        pltpu.sync_copy(idx_hbm.at[wid], idx_v)
        base = pl.multiple_of(wid * BPW * NS, 8)
        bufs = (buf0, buf1)
        off = 0
        writes = []
        gather = None
        for c, n in enumerate(CHUNKS):
            buf = bufs[c % 2]
            if c >= 2:
                writes[c - 2].wait()
            gather = pltpu.async_copy(
                table_hbm.at[idx_v.at[0, pl.ds(off, n)]],
                buf.at[pl.ds(0, n)], gsem)
            gather.wait()
            writes.append(pltpu.async_copy(
                buf.reshape(CHMAX * NS, D).at[pl.ds(0, n * NS)],
                out_hbm.at[pl.ds(base + off * NS, n * NS)], wsem))
            off += n
        writes[-2].wait()
        writes[-1].wait()

    return k(bank3, idx3)


MGRP = 80


def _tc_mask_body(loc_ref, out_ref):
    y = (loc_ref[...] != 0.0).astype(jnp.float32)
    rows = jax.lax.broadcasted_iota(jnp.int32, (MGRP * REP, MGRP), 0)
    cols = jax.lax.broadcasted_iota(jnp.int32, (MGRP * REP, MGRP), 1)
    rep = (rows // REP == cols).astype(jnp.float32)
    out_ref[...] = jnp.dot(rep, y, preferred_element_type=jnp.float32)


def _tc_mask(loc2):
    return pl.pallas_call(
        _tc_mask_body,
        grid=(B * L // MGRP,),
        in_specs=[pl.BlockSpec((MGRP, T), lambda i: (i, 0))],
        out_specs=pl.BlockSpec((MGRP * REP, T), lambda i: (i, 0)),
        out_shape=jax.ShapeDtypeStruct((B * L * REP, T), jnp.float32),
    )(loc2)


def kernel(bank, loc_map, labels, sel_idx):
    bank3 = bank.reshape(C * NQ, NS, D)
    flat_idx = (labels.astype(jnp.int32) * NQ)[:, :, None] + sel_idx.astype(jnp.int32)
    flat_idx = flat_idx.reshape(NROWS)
    rows = _sc_gather_rows(bank3, flat_idx.reshape(NW, 1, BPW))
    batched_queries = rows.reshape(B, L * NQSEL * NS, D)

    batched_mask = _tc_mask(loc_map.reshape(B * L, T)).reshape(B, L * REP, T)

    batched_has_vision_query = jnp.ones((B, L), dtype=jnp.int32)
    return batched_queries, batched_mask, batched_has_vision_query


# speedup vs baseline: 1.9928x; 1.0300x over previous
import functools

import jax
import jax.numpy as jnp
from jax import lax
from jax.experimental import pallas as pl
from jax.experimental.pallas import tpu as pltpu
from jax.experimental.pallas import tpu_sc as plsc

B, L, T = 8, 80, 256
C, NQ, NS, D = 365, 100, 4, 256
NQSEL = 5
RD = NS * D
NROWS = B * L * NQSEL
NW = 32
BPW = NROWS // NW
REP = NQSEL * NS
CHUNKS = (26, 26, 26, 22)
CHMAX = max(CHUNKS)


def _sc_gather_rows(bank3, idx3):
    mesh = plsc.VectorSubcoreMesh(core_axis_name="c", subcore_axis_name="s")

    @functools.partial(
        pl.kernel,
        mesh=mesh,
        out_type=jax.ShapeDtypeStruct((NROWS * NS, D), jnp.float32),
        scratch_types=[
            pltpu.VMEM((1, BPW), jnp.int32),
            pltpu.VMEM((len(CHUNKS), CHMAX, NS, D), jnp.float32),
            pltpu.SemaphoreType.DMA,
            pltpu.SemaphoreType.DMA,
        ],
    )
    def k(table_hbm, idx_hbm, out_hbm, idx_v, bufs, gsem, wsem):
        wid = lax.axis_index("s") * 2 + lax.axis_index("c")
        pltpu.sync_copy(idx_hbm.at[wid], idx_v)
        base = pl.multiple_of(wid * BPW * NS, 8)
        gathers, writes = [], []
        off = 0
        for c, n in enumerate(CHUNKS):
            gathers.append(pltpu.async_copy(
                table_hbm.at[idx_v.at[0, pl.ds(off, n)]],
                bufs.at[c, pl.ds(0, n)], gsem))
            off += n
        off = 0
        for c, n in enumerate(CHUNKS):
            gathers[c].wait()
            writes.append(pltpu.async_copy(
                bufs.at[c].reshape(CHMAX * NS, D).at[pl.ds(0, n * NS)],
                out_hbm.at[pl.ds(base + off * NS, n * NS)], wsem))
            off += n
        for w in writes:
            w.wait()

    return k(bank3, idx3)


MGRP = 80


def _tc_mask_body(loc_ref, out_ref):
    y = (loc_ref[...] != 0.0).astype(jnp.float32)
    rows = jax.lax.broadcasted_iota(jnp.int32, (MGRP * REP, MGRP), 0)
    cols = jax.lax.broadcasted_iota(jnp.int32, (MGRP * REP, MGRP), 1)
    rep = (rows // REP == cols).astype(jnp.float32)
    out_ref[...] = jnp.dot(rep, y, preferred_element_type=jnp.float32)


def _tc_mask(loc2):
    return pl.pallas_call(
        _tc_mask_body,
        grid=(B * L // MGRP,),
        in_specs=[pl.BlockSpec((MGRP, T), lambda i: (i, 0))],
        out_specs=pl.BlockSpec((MGRP * REP, T), lambda i: (i, 0)),
        out_shape=jax.ShapeDtypeStruct((B * L * REP, T), jnp.float32),
    )(loc2)


def kernel(bank, loc_map, labels, sel_idx):
    bank3 = bank.reshape(C * NQ, NS, D)
    flat_idx = (labels.astype(jnp.int32) * NQ)[:, :, None] + sel_idx.astype(jnp.int32)
    flat_idx = flat_idx.reshape(NROWS)
    rows = _sc_gather_rows(bank3, flat_idx.reshape(NW, 1, BPW))
    batched_queries = rows.reshape(B, L * NQSEL * NS, D)

    batched_mask = _tc_mask(loc_map.reshape(B * L, T)).reshape(B, L * REP, T)

    batched_has_vision_query = jnp.ones((B, L), dtype=jnp.int32)
    return batched_queries, batched_mask, batched_has_vision_query


# speedup vs baseline: 1.9951x; 1.0012x over previous
import functools

import jax
import jax.numpy as jnp
from jax import lax
from jax.experimental import pallas as pl
from jax.experimental.pallas import tpu as pltpu
from jax.experimental.pallas import tpu_sc as plsc

B, L, T = 8, 80, 256
C, NQ, NS, D = 365, 100, 4, 256
NQSEL = 5
RD = NS * D
NROWS = B * L * NQSEL
NW = 32
BPW = NROWS // NW
REP = NQSEL * NS
CHUNKS = (26, 26, 26, 22)
CHMAX = max(CHUNKS)


def _sc_gather_rows(bank3, idx3):
    mesh = plsc.VectorSubcoreMesh(core_axis_name="c", subcore_axis_name="s")

    @functools.partial(
        pl.kernel,
        mesh=mesh,
        out_type=jax.ShapeDtypeStruct((NROWS * NS, D), jnp.float32),
        scratch_types=[
            pltpu.VMEM((1, BPW), jnp.int32),
            pltpu.VMEM((len(CHUNKS), CHMAX, NS, D), jnp.float32),
            pltpu.SemaphoreType.DMA,
            pltpu.SemaphoreType.DMA,
        ],
    )
    def k(table_hbm, idx_hbm, out_hbm, idx_v, bufs, gsem, wsem):
        wid = lax.axis_index("s") * 2 + lax.axis_index("c")
        pltpu.sync_copy(idx_hbm.at[wid], idx_v)
        base = pl.multiple_of(wid * BPW * NS, 8)
        gathers, writes = [], []
        off = 0
        for c, n in enumerate(CHUNKS):
            gathers.append(pltpu.async_copy(
                table_hbm.at[idx_v.at[0, pl.ds(off, n)]],
                bufs.at[c, pl.ds(0, n)], gsem))
            off += n
        off = 0
        for c, n in enumerate(CHUNKS):
            gathers[c].wait()
            writes.append(pltpu.async_copy(
                bufs.at[c].reshape(CHMAX * NS, D).at[pl.ds(0, n * NS)],
                out_hbm.at[pl.ds(base + off * NS, n * NS)], wsem))
            off += n
        for w in writes:
            w.wait()

    return k(bank3, idx3)


MGRP = 80


def _tc_mask_body(loc_ref, out_ref):
    y = (loc_ref[...] != 0.0).astype(jnp.bfloat16)
    rows = jax.lax.broadcasted_iota(jnp.int32, (MGRP * REP, MGRP), 0)
    cols = jax.lax.broadcasted_iota(jnp.int32, (MGRP * REP, MGRP), 1)
    rep = (rows // REP == cols).astype(jnp.bfloat16)
    out_ref[...] = jnp.dot(rep, y, preferred_element_type=jnp.float32)


def _tc_mask(loc2):
    return pl.pallas_call(
        _tc_mask_body,
        grid=(B * L // MGRP,),
        in_specs=[pl.BlockSpec((MGRP, T), lambda i: (i, 0))],
        out_specs=pl.BlockSpec((MGRP * REP, T), lambda i: (i, 0)),
        out_shape=jax.ShapeDtypeStruct((B * L * REP, T), jnp.float32),
    )(loc2)


def kernel(bank, loc_map, labels, sel_idx):
    bank3 = bank.reshape(C * NQ, NS, D)
    flat_idx = (labels.astype(jnp.int32) * NQ)[:, :, None] + sel_idx.astype(jnp.int32)
    rows = _sc_gather_rows(bank3, flat_idx.reshape(NW, 1, BPW))
    batched_queries = rows.reshape(B, L * NQSEL * NS, D)

    batched_mask = _tc_mask(loc_map.reshape(B * L, T)).reshape(B, L * REP, T)

    batched_has_vision_query = jnp.ones((B, L), dtype=jnp.int32)
    return batched_queries, batched_mask, batched_has_vision_query
